# Initial kernel scaffold; baseline (speedup 1.0000x reference)
import jax
import jax.numpy as jnp
from jax.experimental import pallas as pl


def kernel(x, edge_index, edge_attr, msg_W_0, msg_b_0, gru_W_ih_0, gru_b_ih_0, gru_W_hh_0, gru_b_hh_0, msg_W_1, msg_b_1, gru_W_ih_1, gru_b_ih_1, gru_W_hh_1, gru_b_hh_1):
    raise NotImplementedError("write your pallas kernel here")


# speedup vs baseline: 5.4014x; 5.4014x over previous
import functools

import jax
import jax.numpy as jnp
from jax import lax
from jax.experimental import pallas as pl
from jax.experimental.pallas import tpu as pltpu
from jax.experimental.pallas import tpu_sc as plsc

NUM_CORES = 2
NUM_SUBCORES = 16
NW = NUM_CORES * NUM_SUBCORES
CHUNK = 128
LANES = 16


@functools.lru_cache(maxsize=None)
def _make_seg_sum(n_acc, chunks, h):
    rpt = n_acc // NUM_SUBCORES

    mesh = plsc.VectorSubcoreMesh(core_axis_name="c", subcore_axis_name="s")

    @functools.partial(
        pl.kernel,
        out_type=jax.ShapeDtypeStruct((NUM_CORES, n_acc, h), jnp.float32),
        mesh=mesh,
        scratch_types=[
            pltpu.VMEM_SHARED((n_acc, h), jnp.float32),
            pltpu.VMEM((2, CHUNK), jnp.int32),
            pltpu.VMEM((CHUNK, h), jnp.float32),
        ],
    )
    def seg(hv_hbm, sidx_hbm, zrow_hbm, out_hbm, acc, idx, rows):
        c = lax.axis_index("c")
        s = lax.axis_index("s")
        wid = s * NUM_CORES + c
        base = s * rpt
        nz = rpt // CHUNK
        pltpu.sync_copy(zrow_hbm, rows)

        def zstep(i, carry):
            pltpu.sync_copy(rows, acc.at[pl.ds(base + i * CHUNK, CHUNK)])
            return carry

        lax.fori_loop(0, nz, zstep, 0)
        plsc.subcore_barrier()

        def step(j, carry):
            pltpu.sync_copy(sidx_hbm.at[wid, j], idx)
            pltpu.sync_copy(hv_hbm.at[idx.at[0]], rows)
            pltpu.sync_copy(rows, acc.at[idx.at[1]], add=True)
            return carry

        lax.fori_loop(0, chunks, step, 0)
        plsc.subcore_barrier()

        def ostep(i, carry):
            off = base + i * CHUNK
            pltpu.sync_copy(acc.at[pl.ds(off, CHUNK)], rows)
            pltpu.sync_copy(rows, out_hbm.at[c, pl.ds(off, CHUNK)])
            return carry

        lax.fori_loop(0, nz, ostep, 0)

    return seg


@functools.lru_cache(maxsize=None)
def _make_aux(n_acc, chunks, h):
    rpt = n_acc // NUM_SUBCORES

    mesh = plsc.VectorSubcoreMesh(core_axis_name="c", subcore_axis_name="s")

    @functools.partial(
        pl.kernel,
        out_type=jax.ShapeDtypeStruct((NUM_CORES, n_acc, h), jnp.float32),
        mesh=mesh,
        scratch_types=[
            pltpu.VMEM_SHARED((n_acc, h), jnp.float32),
            pltpu.VMEM((2, CHUNK), jnp.int32),
            pltpu.VMEM((CHUNK // LANES, LANES), jnp.float32),
            pltpu.VMEM((CHUNK, h), jnp.float32),
        ],
    )
    def aux(ea_hbm, sidx_hbm, zrow_hbm, out_hbm, acc, idx, eab, rows):
        c = lax.axis_index("c")
        s = lax.axis_index("s")
        wid = s * NUM_CORES + c
        base = s * rpt
        nz = rpt // CHUNK
        pltpu.sync_copy(zrow_hbm, rows)

        def zstep(i, carry):
            pltpu.sync_copy(rows, acc.at[pl.ds(base + i * CHUNK, CHUNK)])
            return carry

        lax.fori_loop(0, nz, zstep, 0)
        plsc.subcore_barrier()

        lanes = lax.iota(jnp.int32, LANES)
        one = jnp.ones((LANES,), jnp.float32)
        zero = jnp.zeros((LANES,), jnp.float32)
        for g in range(CHUNK // LANES):
            for i in range(LANES):
                rows[g * LANES + i, pl.ds(LANES, LANES)] = jnp.where(
                    lanes == i, one, zero)

        def step(j, carry):
            pltpu.sync_copy(sidx_hbm.at[wid, j], idx)
            pltpu.sync_copy(ea_hbm.at[wid, j], eab)
            for g in range(CHUNK // LANES):
                eav = eab[g]
                for i in range(LANES):
                    rows[g * LANES + i, pl.ds(0, LANES)] = jnp.where(
                        lanes == i, eav, zero)
            pltpu.sync_copy(rows, acc.at[idx.at[1]], add=True)
            return carry

        lax.fori_loop(0, chunks, step, 0)
        plsc.subcore_barrier()

        def ostep(i, carry):
            off = base + i * CHUNK
            pltpu.sync_copy(acc.at[pl.ds(off, CHUNK)], rows)
            pltpu.sync_copy(rows, out_hbm.at[c, pl.ds(off, CHUNK)])
            return carry

        lax.fori_loop(0, nz, ostep, 0)

    return aux


def _dense_body(hv_ref, s0_ref, s1_ref, a0_ref, a1_ref, wd_ref, ws_ref,
                we_ref, mb_ref, wih_ref, bih_ref, whh_ref, bhh_ref, out_ref):
    h = hv_ref.shape[1]
    hv = hv_ref[...]
    s_sum = s0_ref[0] + s1_ref[0]
    auxp = a0_ref[0] + a1_ref[0]
    col = lax.broadcasted_iota(jnp.int32, (1, h), 1)
    ea = jnp.sum(jnp.where(col < LANES, auxp, 0.0), axis=1, keepdims=True)
    deg = jnp.sum(jnp.where((col >= LANES) & (col < 2 * LANES), auxp, 0.0),
                  axis=1, keepdims=True)
    f32 = jnp.float32
    hi = lax.Precision.HIGHEST
    hv_r = hv.astype(jnp.bfloat16).astype(f32)
    a = deg * jnp.dot(hv_r, wd_ref[...], preferred_element_type=f32,
                      precision=hi)
    a = a + jnp.dot(s_sum, ws_ref[...], preferred_element_type=f32,
                    precision=hi)
    a = a + ea * we_ref[...] + deg * mb_ref[...]
    a_r = a.astype(jnp.bfloat16).astype(f32)
    gi = jnp.dot(a_r, wih_ref[...], preferred_element_type=f32,
                 precision=hi) + bih_ref[...]
    gh = jnp.dot(hv_r, whh_ref[...], preferred_element_type=f32,
                 precision=hi) + bhh_ref[...]
    r = jax.nn.sigmoid(gi[:, :h] + gh[:, :h])
    z = jax.nn.sigmoid(gi[:, h:2 * h] + gh[:, h:2 * h])
    n = jnp.tanh(gi[:, 2 * h:] + r * gh[:, 2 * h:])
    out_ref[...] = (1.0 - z) * n + z * hv


def _dense_update(hv, s_parts, a_parts, wd, ws, we, mb, wih, bih, whh, bhh,
                  block):
    n, h = hv.shape
    grid = (n // block,)
    row = lambda i: (i, 0)
    rep = lambda i: (0, 0)
    part0 = pl.BlockSpec((1, block, h), lambda i: (0, i, 0))
    part1 = pl.BlockSpec((1, block, h), lambda i: (1, i, 0))
    return pl.pallas_call(
        _dense_body,
        grid=grid,
        in_specs=[
            pl.BlockSpec((block, h), row),
            part0, part1, part0, part1,
            pl.BlockSpec(wd.shape, rep),
            pl.BlockSpec(ws.shape, rep),
            pl.BlockSpec(we.shape, rep),
            pl.BlockSpec(mb.shape, rep),
            pl.BlockSpec(wih.shape, rep),
            pl.BlockSpec(bih.shape, rep),
            pl.BlockSpec(whh.shape, rep),
            pl.BlockSpec(bhh.shape, rep),
        ],
        out_specs=pl.BlockSpec((block, h), row),
        out_shape=jax.ShapeDtypeStruct((n, h), jnp.float32),
    )(hv, s_parts, s_parts, a_parts, a_parts,
      wd, ws, we, mb, wih, bih, whh, bhh)


def kernel(x, edge_index, edge_attr, msg_W_0, msg_b_0, gru_W_ih_0,
           gru_b_ih_0, gru_W_hh_0, gru_b_hh_0, msg_W_1, msg_b_1, gru_W_ih_1,
           gru_b_ih_1, gru_W_hh_1, gru_b_hh_1):
    n, h = x.shape
    e = edge_index.shape[1]
    per_worker = -(-e // (NW * CHUNK)) * CHUNK
    ep = per_worker * NW
    chunks = per_worker // CHUNK
    n_acc = -(-(n + 1) // (NUM_SUBCORES * CHUNK)) * (NUM_SUBCORES * CHUNK)
    block = 1000

    src = edge_index[0]
    dst = edge_index[1]
    pad = ep - e
    src_p = jnp.concatenate([src, jnp.zeros((pad,), jnp.int32)])
    dst_p = jnp.concatenate([dst, jnp.full((pad,), n, jnp.int32)])
    ea_p = jnp.concatenate([
        edge_attr[:, 0].astype(jnp.bfloat16).astype(jnp.float32),
        jnp.zeros((pad,), jnp.float32)])
    ea4 = ea_p.reshape(NW, chunks, CHUNK // LANES, LANES)
    sidx = jnp.stack([src_p.reshape(NW, chunks, CHUNK),
                      dst_p.reshape(NW, chunks, CHUNK)], axis=2)
    zrow = jnp.zeros((CHUNK, h), jnp.float32)

    seg = _make_seg_sum(n_acc, chunks, h)
    aux = _make_aux(n_acc, chunks, h)
    a_parts = aux(ea4, sidx, zrow)

    params = (
        (msg_W_0, msg_b_0, gru_W_ih_0, gru_b_ih_0, gru_W_hh_0, gru_b_hh_0),
        (msg_W_1, msg_b_1, gru_W_ih_1, gru_b_ih_1, gru_W_hh_1, gru_b_hh_1),
    )
    def bf(w):
        return w.astype(jnp.bfloat16).astype(jnp.float32)

    hv = x
    for (mw, mb, wih, bih, whh, bhh) in params:
        hv_r = bf(hv)
        s_parts = seg(hv_r, sidx, zrow)
        hv = _dense_update(
            hv, s_parts, a_parts,
            bf(mw[:, :h].T), bf(mw[:, h:2 * h].T),
            bf(mw[:, 2 * h][None, :]), mb[None, :],
            bf(wih.T), bih[None, :], bf(whh.T), bhh[None, :],
            block,
        )
    return hv
